# baseline (speedup 1.0000x reference)
import functools

import jax
import jax.numpy as jnp
from jax import lax
from jax.experimental import pallas as pl
from jax.experimental.pallas import tpu as pltpu
from jax.experimental.pallas import tpu_sc as plsc

_NC = 2
_NS = 16
_NW = _NC * _NS
_L = 16


@functools.lru_cache(maxsize=None)
def _build(H: int, B: int, D: int, V: int):
    assert D % _NW == 0 and B % _L == 0
    d_per_w = D // _NW
    mesh = plsc.VectorSubcoreMesh(core_axis_name="c", subcore_axis_name="s")

    @functools.partial(
        pl.kernel,
        mesh=mesh,
        compiler_params=pltpu.CompilerParams(
            use_tc_tiling_on_sc=True, needs_layout_passes=False
        ),
        out_type=jax.ShapeDtypeStruct((H, D, B), jnp.float32),
        scratch_types=[
            pltpu.VMEM((V,), jnp.float32),
            [pltpu.VMEM((B,), jnp.int32) for _ in range(2)],
            [pltpu.VMEM((B,), jnp.float32) for _ in range(2)],
            [pltpu.SemaphoreType.DMA for _ in range(2)],
            [pltpu.SemaphoreType.DMA for _ in range(2)],
        ],
    )
    def gather(x_hbm, table_hbm, out_hbm, row_v, idx_v, res_v, sem_i, sem_o):
        sid = lax.axis_index("s")
        wid = sid * _NC + lax.axis_index("c")

        def one_dim(d, first, last):
            pltpu.sync_copy(table_hbm.at[d], row_v)
            if first:
                pltpu.async_copy(x_hbm.at[0], idx_v[0], sem_i[0])

            def do_h(h, p):
                pltpu.make_async_copy(x_hbm.at[h], idx_v[p], sem_i[p]).wait()

                @pl.when(h + 1 < H)
                def _():
                    pltpu.async_copy(x_hbm.at[h + 1], idx_v[1 - p], sem_i[1 - p])

                @pl.when(h >= 2)
                def _():
                    pltpu.make_async_copy(
                        res_v[p], out_hbm.at[h - 2, d], sem_o[p]
                    ).wait()

                @plsc.parallel_loop(0, B, step=_L, unroll=16)
                def _(i):
                    idx = idx_v[p][pl.ds(i, _L)]
                    res_v[p][pl.ds(i, _L)] = plsc.load_gather(row_v, [idx])

                pltpu.async_copy(res_v[p], out_hbm.at[h, d], sem_o[p])

            def pair(o, carry):
                do_h(2 * o, 0)
                do_h(2 * o + 1, 1)
                return carry

            lax.fori_loop(0, H // 2, pair, 0)
            for p in range(2):
                pltpu.make_async_copy(
                    res_v[p], out_hbm.at[H - 2 + p, d], sem_o[p]
                ).wait()
            if not last:
                pltpu.async_copy(x_hbm.at[0], idx_v[0], sem_i[0])

        for dd in range(d_per_w):
            one_dim(wid * d_per_w + dd, dd == 0, dd == d_per_w - 1)

    return gather


def kernel(x, table):
    B, H = x.shape
    V, D = table.shape
    out_t = _build(H, B, D, V)(x.T, table.T.astype(jnp.float32))
    return out_t.transpose(2, 0, 1)


# speedup vs baseline: 1.1469x; 1.1469x over previous
import functools

import jax
import jax.numpy as jnp
from jax import lax
from jax.experimental import pallas as pl
from jax.experimental.pallas import tpu as pltpu
from jax.experimental.pallas import tpu_sc as plsc

_NC = 2
_NS = 16
_NW = _NC * _NS
_L = 16
_NRES = 3


@functools.lru_cache(maxsize=None)
def _build(H: int, B: int, D: int, V: int):
    assert D % _NW == 0 and B % _L == 0 and H % 2 == 0
    d_per_w = D // _NW
    mesh = plsc.VectorSubcoreMesh(core_axis_name="c", subcore_axis_name="s")

    @functools.partial(
        pl.kernel,
        mesh=mesh,
        compiler_params=pltpu.CompilerParams(
            use_tc_tiling_on_sc=True, needs_layout_passes=False
        ),
        out_type=jax.ShapeDtypeStruct((H, D, B), jnp.float32),
        scratch_types=[
            pltpu.VMEM((V,), jnp.float32),
            [pltpu.VMEM((2, B), jnp.int32) for _ in range(2)],
            [pltpu.VMEM((B,), jnp.float32) for _ in range(_NRES)],
            [pltpu.SemaphoreType.DMA for _ in range(2)],
            [pltpu.SemaphoreType.DMA for _ in range(_NRES)],
        ],
    )
    def gather(x_hbm, table_hbm, out_hbm, row_v, idx_v, res_v, sem_i, sem_o):
        wid = lax.axis_index("s") * _NC + lax.axis_index("c")

        def one_dim(d, first, last):
            pltpu.sync_copy(table_hbm.at[d], row_v)
            if first:
                pltpu.async_copy(x_hbm.at[pl.ds(0, 2)], idx_v[0], sem_i[0])

            def do_pair(g, q):
                pltpu.make_async_copy(
                    x_hbm.at[pl.ds(2 * g, 2)], idx_v[q], sem_i[q]
                ).wait()

                if g + 1 < H // 2:
                    pltpu.async_copy(
                        x_hbm.at[pl.ds(2 * g + 2, 2)], idx_v[1 - q],
                        sem_i[1 - q],
                    )

                for j in range(2):
                    h = 2 * g + j
                    r = (2 * g + j) % _NRES

                    if h >= _NRES:
                        pltpu.make_async_copy(
                            res_v[r], out_hbm.at[h - _NRES, d], sem_o[r]
                        ).wait()

                    @plsc.parallel_loop(0, B, step=_L, unroll=8)
                    def _(i):
                        idx = idx_v[q][j, pl.ds(i, _L)]
                        res_v[r][pl.ds(i, _L)] = plsc.load_gather(
                            row_v, [idx]
                        )

                    pltpu.async_copy(res_v[r], out_hbm.at[h, d], sem_o[r])

            for g in range(H // 2):
                do_pair(g, g % 2)
            for k in range(_NRES):
                h = H - _NRES + k
                pltpu.make_async_copy(
                    res_v[h % _NRES], out_hbm.at[h, d], sem_o[h % _NRES]
                ).wait()
            if not last:
                pltpu.async_copy(x_hbm.at[pl.ds(0, 2)], idx_v[0], sem_i[0])

        for dd in range(d_per_w):
            one_dim(wid * d_per_w + dd, dd == 0, dd == d_per_w - 1)

    return gather


def kernel(x, table):
    B, H = x.shape
    V, D = table.shape
    out_t = _build(H, B, D, V)(x.T, table.T)
    return out_t.transpose(2, 0, 1)
